# baseline (speedup 1.0000x reference)
import functools
import jax
import jax.numpy as jnp
from jax import lax
from jax.experimental import pallas as pl
from jax.experimental.pallas import tpu as pltpu
from jax.experimental.pallas import tpu_sc as plsc

N_ENT = 4096
N_REL = 200
DIM = 64
B = 128
NIDX = 4 * B
ALPHA = 0.1
MARGIN = 1.0

SC_CORES = 2
SC_SUBCORES = 16
NW = SC_CORES * SC_SUBCORES
RPW = NIDX // NW
LANES = 16
GROUPS = NIDX // LANES


def _sc_gather_body(p_hbm, idx1_hbm, out_hbm, idx1c_v, rows_v, sem):
    wid = lax.axis_index("s") * SC_CORES + lax.axis_index("c")
    base = wid * RPW
    pltpu.sync_copy(idx1_hbm.at[pl.ds(base, RPW)], idx1c_v)
    pltpu.async_copy(p_hbm.at[idx1c_v], rows_v, sem).wait()
    pltpu.sync_copy(rows_v, out_hbm.at[pl.ds(base, RPW)])


@functools.lru_cache(maxsize=None)
def _make_sc_gather():
    return functools.partial(
        pl.kernel,
        out_type=jax.ShapeDtypeStruct((NIDX, N_ENT), jnp.float32),
        mesh=plsc.VectorSubcoreMesh(core_axis_name="c", subcore_axis_name="s"),
        scratch_types=[
            pltpu.VMEM((RPW,), jnp.int32),
            pltpu.VMEM((RPW, N_ENT), jnp.float32),
            pltpu.SemaphoreType.DMA,
        ],
    )(_sc_gather_body)


def _onehot_rows(idx_col, n_cols):
    r = idx_col.shape[0]
    cols = lax.broadcasted_iota(jnp.int32, (r, n_cols), 1)
    return jnp.where(cols == idx_col, jnp.float32(1.0), jnp.float32(0.0))


def _rescal_losses(ent, relf, h_i, t_i, nh_i, nt_i, r_i):
    h = jnp.dot(_onehot_rows(h_i, N_ENT), ent, preferred_element_type=jnp.float32)
    t = jnp.dot(_onehot_rows(t_i, N_ENT), ent, preferred_element_type=jnp.float32)
    nh = jnp.dot(_onehot_rows(nh_i, N_ENT), ent, preferred_element_type=jnp.float32)
    nt = jnp.dot(_onehot_rows(nt_i, N_ENT), ent, preferred_element_type=jnp.float32)
    rg = jnp.dot(_onehot_rows(r_i, N_REL), relf, preferred_element_type=jnp.float32)

    rows64 = lax.broadcasted_iota(jnp.int32, (DIM, DIM * DIM), 0)
    colmod = lax.broadcasted_iota(jnp.int32, (DIM, DIM * DIM), 1) % DIM
    tile_m = jnp.where(colmod == rows64, jnp.float32(1.0), jnp.float32(0.0))
    segrows = lax.broadcasted_iota(jnp.int32, (DIM * DIM, DIM), 0) // DIM
    segcols = lax.broadcasted_iota(jnp.int32, (DIM * DIM, DIM), 1)
    seg_m = jnp.where(segrows == segcols, jnp.float32(1.0), jnp.float32(0.0))

    def score(hv, tv):
        trep = jnp.dot(tv, tile_m, preferred_element_type=jnp.float32)
        tmp = jnp.dot(rg * trep, seg_m, preferred_element_type=jnp.float32)
        return jnp.sum(hv * tmp, axis=1)

    pos = score(h, t)
    neg = score(nh, nt)
    return jnp.mean(jax.nn.relu(MARGIN + neg - pos))


def _tc_body(g_rows, ent0, ent1, rel0f, rel1f, idx1c, idx2c,
             h0, t0, nh0, nt0, r0, h1, t1, nh1, nt1, r1, out):
    e1 = jnp.dot(_onehot_rows(idx1c[...], N_ENT), ent0[...],
                 preferred_element_type=jnp.float32)
    e2 = jnp.dot(_onehot_rows(idx2c[...], N_ENT), ent1[...],
                 preferred_element_type=jnp.float32)
    a = jnp.sum(e1 * e1, axis=1, keepdims=True)
    b = jnp.sum(e2 * e2, axis=1, keepdims=True)
    ones = jnp.ones((NIDX, 1), jnp.float32)
    zeros = jnp.zeros((NIDX, 128 - 2 - DIM), jnp.float32)
    c_mat = jnp.concatenate([ones, b, e2, zeros], axis=1)
    u_mat = jnp.concatenate([a, ones, -2.0 * e1, zeros], axis=1)
    rows_iota = lax.broadcasted_iota(jnp.int32, (N_ENT, NIDX), 0)
    o2t = jnp.where(rows_iota == jnp.reshape(idx2c[...], (1, NIDX)),
                    jnp.float32(1.0), jnp.float32(0.0))
    w_mat = jnp.dot(o2t, c_mat, preferred_element_type=jnp.float32)
    m = jnp.dot(g_rows[...], w_mat, preferred_element_type=jnp.float32)
    ot = jnp.sum(m * u_mat)

    l0 = _rescal_losses(ent0[...], rel0f[...], h0[...], t0[...], nh0[...], nt0[...], r0[...])
    l1 = _rescal_losses(ent1[...], rel1f[...], h1[...], t1[...], nh1[...], nt1[...], r1[...])

    lane = lax.broadcasted_iota(jnp.int32, (1, 128), 1)
    out[...] = jnp.where(lane == 0, l0,
                         jnp.where(lane == 1, l1,
                                   jnp.where(lane == 2, ALPHA * ot, 0.0)))


@jax.jit
def kernel(heads_0, tails_0, n_heads_0, n_tails_0, rels_0,
           heads_1, tails_1, n_heads_1, n_tails_1, rels_1,
           ent_emb_0, rel_emb_0, ent_emb_1, rel_emb_1, P):
    idx1 = jnp.concatenate([heads_0, tails_0, n_heads_0, n_tails_0]).astype(jnp.int32)
    idx2 = jnp.concatenate([heads_1, tails_1, n_heads_1, n_tails_1]).astype(jnp.int32)
    col = lambda x: jnp.reshape(x.astype(jnp.int32), (-1, 1))
    rel0f = jnp.reshape(rel_emb_0, (N_REL, DIM * DIM))
    rel1f = jnp.reshape(rel_emb_1, (N_REL, DIM * DIM))

    g_rows = _make_sc_gather()(P, idx1)

    vmem = pl.BlockSpec(memory_space=pltpu.VMEM)
    out = pl.pallas_call(
        _tc_body,
        in_specs=[vmem] * 17,
        out_specs=vmem,
        out_shape=jax.ShapeDtypeStruct((1, 128), jnp.float32),
    )(g_rows, ent_emb_0, ent_emb_1, rel0f, rel1f, col(idx1), col(idx2),
      col(heads_0), col(tails_0), col(n_heads_0), col(n_tails_0), col(rels_0),
      col(heads_1), col(tails_1), col(n_heads_1), col(n_tails_1), col(rels_1))
    return (out[0, :2], out[0, 2])
